# baseline (speedup 1.0000x reference)
import functools

import jax
import jax.numpy as jnp
from jax import lax
from jax.experimental import pallas as pl
from jax.experimental.pallas import tpu as pltpu
from jax.experimental.pallas import tpu_sc as plsc

B, L, D = 4, 8192, 1024
MAXC = L // 4
NSEG = 8
SEG = L // NSEG
OUTSEG = MAXC // NSEG
NCHUNK = 8
CROWS = OUTSEG // NCHUNK


def _ids():
  c = lax.axis_index("c")
  s = lax.axis_index("s")
  r = 2 * c + s // NSEG
  seg = s % NSEG
  lane = lax.broadcasted_iota(jnp.int32, (16,), 0)
  return c, s, r, seg, lane


def _group_counts(cnt_loc, s, lane):
  cnts = jnp.zeros((16,), jnp.int32)
  for l in range(16):
    cnts = jnp.where(lane == l, cnt_loc[l, pl.ds(0, 16)], cnts)
  g0 = (s // NSEG) * NSEG
  grp = (lane >= g0) & (lane < g0 + NSEG)
  num_true = jnp.sum(jnp.where(grp, cnts, 0))
  pre = jnp.sum(jnp.where(grp & (lane < s), cnts, 0))
  return num_true, pre


def _compact_body(mask_hbm, takef_hbm, pmask_hbm, cnt_hbm,
                  mseg, destb, valb, cnt_tmp, cnt_loc, pmb, sem_scat):
  c, s, r, seg, lane = _ids()
  seg_base = seg * SEG
  rb = r * L

  pltpu.sync_copy(mask_hbm.at[r, pl.ds(seg_base, SEG)], mseg)

  def cbody(i, acc):
    mv = mseg[pl.ds(i * 16, 16)]
    return acc + (mv != 0).astype(jnp.int32)

  acc = lax.fori_loop(0, SEG // 16, cbody, jnp.zeros((16,), jnp.int32))
  lt = jnp.sum(acc)

  cnt_tmp[...] = jnp.broadcast_to(lt, (16,))
  pltpu.sync_copy(cnt_tmp, cnt_hbm.at[c * 16 + s])
  plsc.subcore_barrier()
  pltpu.sync_copy(cnt_hbm.at[pl.ds(c * 16, 16)], cnt_loc)
  num_true, pre = _group_counts(cnt_loc, s, lane)
  num_chunks = jnp.minimum(num_true, MAXC)

  tc = rb + pre
  fc = rb + num_true + (seg_base - pre)
  for i in range(SEG // 16):
    mv = mseg[pl.ds(i * 16, 16)]
    mi = (mv != 0).astype(jnp.int32)
    incl = plsc.cumsum(mi)
    rank = incl - mi
    dest = jnp.where(mv != 0, tc + rank, fc + (lane - rank))
    dest = jnp.clip(dest, rb, rb + L - 1)
    destb[i // 8, pl.ds((i % 8) * 16, 16)] = dest
    valb[i // 8, pl.ds((i % 8) * 16, 16)] = seg_base + i * 16 + lane
    cnt = jnp.max(incl)
    tc = tc + cnt
    fc = fc + (16 - cnt)

  handles = []
  for q in range(NCHUNK):
    handles.append(
        pltpu.async_copy(valb.at[q], takef_hbm.at[destb.at[q]], sem_scat))
  for h in handles:
    h.wait()

  for i in range(OUTSEG // 16):
    pos = seg * OUTSEG + i * 16 + lane
    pmb[pl.ds(i * 16, 16)] = jnp.where(pos < num_chunks, 1, 0)
  pltpu.sync_copy(pmb, pmask_hbm.at[r, pl.ds(seg * OUTSEG, OUTSEG)])


def _gather_body(hs_hbm, takef_hbm, cnt_hbm, chunked_hbm,
                 cnt_loc, idx1, idx2, inbuf0, inbuf1,
                 sem_in, sem_out0, sem_out1):
  c, s, r, seg, lane = _ids()
  rb = r * L

  pltpu.sync_copy(cnt_hbm.at[pl.ds(c * 16, 16)], cnt_loc)
  num_true, _ = _group_counts(cnt_loc, s, lane)
  num_chunks = jnp.minimum(num_true, MAXC)

  idx_base = rb + seg * OUTSEG
  pltpu.sync_copy(takef_hbm.at[pl.ds(idx_base, OUTSEG)], idx1)
  for k in range(NCHUNK):
    for w in range(CROWS // 16):
      idx2[k, pl.ds(w * 16, 16)] = jnp.clip(
          idx1[pl.ds(k * CROWS + w * 16, 16)] + rb, 0, B * L - 1)

  zs = jnp.clip(num_chunks - seg * OUTSEG, 0, OUTSEG)
  crow_base = r * MAXC + seg * OUTSEG
  out_handles = [None, None]
  for k in range(NCHUNK):
    buf = inbuf0 if k % 2 == 0 else inbuf1
    if k >= 2:
      out_handles[k % 2].wait()
    pltpu.async_copy(hs_hbm.at[idx2.at[k]], buf, sem_in).wait()
    zlo = jnp.clip(zs - k * CROWS, 0, CROWS)

    @pl.when(zlo < CROWS)
    def _zero(buf=buf, zlo=zlo):
      def zbody(row, carry):
        for w in range(D // 16):
          buf[row, pl.ds(w * 16, 16)] = jnp.zeros((16,), jnp.float32)
        return carry
      lax.fori_loop(zlo, CROWS, zbody, 0)

    sem_out = sem_out0 if k % 2 == 0 else sem_out1
    out_handles[k % 2] = pltpu.async_copy(
        buf, chunked_hbm.at[pl.ds(crow_base + k * CROWS, CROWS)], sem_out)
  out_handles[0].wait()
  out_handles[1].wait()


_MESH = plsc.VectorSubcoreMesh(core_axis_name="c", subcore_axis_name="s",
                               num_cores=2, num_subcores=16)

_compact_call = functools.partial(
    pl.kernel,
    out_type=[
        jax.ShapeDtypeStruct((B * L,), jnp.int32),
        jax.ShapeDtypeStruct((B, MAXC), jnp.int32),
        jax.ShapeDtypeStruct((32, 16), jnp.int32),
    ],
    mesh=_MESH,
    compiler_params=pltpu.CompilerParams(needs_layout_passes=False),
    scratch_types=[
        pltpu.VMEM((SEG,), jnp.int32),
        pltpu.VMEM((NCHUNK, 128), jnp.int32),
        pltpu.VMEM((NCHUNK, 128), jnp.int32),
        pltpu.VMEM((16,), jnp.int32),
        pltpu.VMEM((16, 16), jnp.int32),
        pltpu.VMEM((OUTSEG,), jnp.int32),
        pltpu.SemaphoreType.DMA,
    ],
)(_compact_body)

_gather_call = functools.partial(
    pl.kernel,
    out_type=[
        jax.ShapeDtypeStruct((B * MAXC, D), jnp.float32),
    ],
    mesh=_MESH,
    compiler_params=pltpu.CompilerParams(needs_layout_passes=False),
    scratch_types=[
        pltpu.VMEM((16, 16), jnp.int32),
        pltpu.VMEM((OUTSEG,), jnp.int32),
        pltpu.VMEM((NCHUNK, CROWS), jnp.int32),
        pltpu.VMEM((CROWS, D), jnp.float32),
        pltpu.VMEM((CROWS, D), jnp.float32),
        pltpu.SemaphoreType.DMA,
        pltpu.SemaphoreType.DMA,
        pltpu.SemaphoreType.DMA,
    ],
)(_gather_body)


@jax.jit
def kernel(hidden_states, boundary_mask):
  hs_flat = hidden_states.reshape(B * L, D)
  mask_i32 = boundary_mask.astype(jnp.int32)
  takef, pm, cnt = _compact_call(mask_i32)
  (chunked_flat,) = _gather_call(hs_flat, takef, cnt)
  chunked = chunked_flat.reshape(B, MAXC, D)
  take_idx = takef.reshape(B, L)[:, :MAXC]
  pad_mask = pm.astype(jnp.bool_)
  return chunked, pad_mask, take_idx


# speedup vs baseline: 1.0117x; 1.0117x over previous
import functools

import jax
import jax.numpy as jnp
from jax import lax
from jax.experimental import pallas as pl
from jax.experimental.pallas import tpu as pltpu
from jax.experimental.pallas import tpu_sc as plsc

B, L, D = 4, 8192, 1024
MAXC = L // 4
NSEG = 8
SEG = L // NSEG
OUTSEG = MAXC // NSEG
NCHUNK = 8
CROWS = OUTSEG // NCHUNK


def _ids():
  c = lax.axis_index("c")
  s = lax.axis_index("s")
  r = 2 * c + s // NSEG
  seg = s % NSEG
  lane = lax.broadcasted_iota(jnp.int32, (16,), 0)
  return c, s, r, seg, lane


def _group_counts(cnt_loc, s, lane):
  cnts = jnp.zeros((16,), jnp.int32)
  for l in range(16):
    cnts = jnp.where(lane == l, cnt_loc[l, pl.ds(0, 16)], cnts)
  g0 = (s // NSEG) * NSEG
  grp = (lane >= g0) & (lane < g0 + NSEG)
  num_true = jnp.sum(jnp.where(grp, cnts, 0))
  pre = jnp.sum(jnp.where(grp & (lane < s), cnts, 0))
  return num_true, pre


def _compact_body(mask_hbm, takef_hbm, pmask_hbm, cnt_hbm,
                  mseg, destb, valb, cnt_tmp, cnt_loc, pmb, sem_scat):
  c, s, r, seg, lane = _ids()
  seg_base = seg * SEG
  rb = r * L

  pltpu.sync_copy(mask_hbm.at[r, pl.ds(seg_base, SEG)], mseg)

  def cbody(i, acc):
    mv = mseg[pl.ds(i * 16, 16)]
    return acc + (mv != 0).astype(jnp.int32)

  acc = lax.fori_loop(0, SEG // 16, cbody, jnp.zeros((16,), jnp.int32))
  lt = jnp.sum(acc)

  cnt_tmp[...] = jnp.broadcast_to(lt, (16,))
  pltpu.sync_copy(cnt_tmp, cnt_hbm.at[c * 16 + s])
  plsc.subcore_barrier()
  pltpu.sync_copy(cnt_hbm.at[pl.ds(c * 16, 16)], cnt_loc)
  num_true, pre = _group_counts(cnt_loc, s, lane)
  num_chunks = jnp.minimum(num_true, MAXC)

  tc = rb + pre
  fc = rb + num_true + (seg_base - pre)
  for i in range(SEG // 16):
    mv = mseg[pl.ds(i * 16, 16)]
    mi = (mv != 0).astype(jnp.int32)
    incl = plsc.cumsum(mi)
    rank = incl - mi
    dest = jnp.where(mv != 0, tc + rank, fc + (lane - rank))
    dest = jnp.clip(dest, rb, rb + L - 1)
    destb[i // 8, pl.ds((i % 8) * 16, 16)] = dest
    valb[i // 8, pl.ds((i % 8) * 16, 16)] = seg_base + i * 16 + lane
    cnt = jnp.max(incl)
    tc = tc + cnt
    fc = fc + (16 - cnt)

  handles = []
  for q in range(NCHUNK):
    handles.append(
        pltpu.async_copy(valb.at[q], takef_hbm.at[destb.at[q]], sem_scat))
  for h in handles:
    h.wait()

  for i in range(OUTSEG // 16):
    pos = seg * OUTSEG + i * 16 + lane
    pmb[pl.ds(i * 16, 16)] = jnp.where(pos < num_chunks, 1, 0)
  pltpu.sync_copy(pmb, pmask_hbm.at[r, pl.ds(seg * OUTSEG, OUTSEG)])


def _gather_body(hs_hbm, takef_hbm, cnt_hbm, chunked_hbm,
                 cnt_loc, idx1, idx2, inbuf0, inbuf1, inbuf2,
                 gs0, gs1, gs2, os0, os1, os2):
  c, s, r, seg, lane = _ids()
  rb = r * L

  pltpu.sync_copy(cnt_hbm.at[pl.ds(c * 16, 16)], cnt_loc)
  num_true, _ = _group_counts(cnt_loc, s, lane)
  num_chunks = jnp.minimum(num_true, MAXC)

  idx_base = rb + seg * OUTSEG
  pltpu.sync_copy(takef_hbm.at[pl.ds(idx_base, OUTSEG)], idx1)
  for k in range(NCHUNK):
    for w in range(CROWS // 16):
      idx2[k, pl.ds(w * 16, 16)] = jnp.clip(
          idx1[pl.ds(k * CROWS + w * 16, 16)] + rb, 0, B * L - 1)

  zs = jnp.clip(num_chunks - seg * OUTSEG, 0, OUTSEG)
  crow_base = r * MAXC + seg * OUTSEG
  bufs = (inbuf0, inbuf1, inbuf2)
  gsems = (gs0, gs1, gs2)
  osems = (os0, os1, os2)
  gh = [None] * NCHUNK
  oh = [None] * NCHUNK
  for k in range(2):
    gh[k] = pltpu.async_copy(hs_hbm.at[idx2.at[k]], bufs[k % 3], gsems[k % 3])
  for k in range(NCHUNK):
    buf = bufs[k % 3]
    gh[k].wait()
    zlo = jnp.clip(zs - k * CROWS, 0, CROWS)

    @pl.when(zlo < CROWS)
    def _zero(buf=buf, zlo=zlo):
      def zbody(row, carry):
        for w in range(D // 16):
          buf[row, pl.ds(w * 16, 16)] = jnp.zeros((16,), jnp.float32)
        return carry
      lax.fori_loop(zlo, CROWS, zbody, 0)

    oh[k] = pltpu.async_copy(
        buf, chunked_hbm.at[pl.ds(crow_base + k * CROWS, CROWS)], osems[k % 3])
    nk = k + 2
    if nk < NCHUNK:
      if nk >= 3:
        oh[nk - 3].wait()
      gh[nk] = pltpu.async_copy(
          hs_hbm.at[idx2.at[nk]], bufs[nk % 3], gsems[nk % 3])
  oh[NCHUNK - 3].wait()
  oh[NCHUNK - 2].wait()
  oh[NCHUNK - 1].wait()


_MESH = plsc.VectorSubcoreMesh(core_axis_name="c", subcore_axis_name="s",
                               num_cores=2, num_subcores=16)

_compact_call = functools.partial(
    pl.kernel,
    out_type=[
        jax.ShapeDtypeStruct((B * L,), jnp.int32),
        jax.ShapeDtypeStruct((B, MAXC), jnp.int32),
        jax.ShapeDtypeStruct((32, 16), jnp.int32),
    ],
    mesh=_MESH,
    compiler_params=pltpu.CompilerParams(needs_layout_passes=False),
    scratch_types=[
        pltpu.VMEM((SEG,), jnp.int32),
        pltpu.VMEM((NCHUNK, 128), jnp.int32),
        pltpu.VMEM((NCHUNK, 128), jnp.int32),
        pltpu.VMEM((16,), jnp.int32),
        pltpu.VMEM((16, 16), jnp.int32),
        pltpu.VMEM((OUTSEG,), jnp.int32),
        pltpu.SemaphoreType.DMA,
    ],
)(_compact_body)

_gather_call = functools.partial(
    pl.kernel,
    out_type=[
        jax.ShapeDtypeStruct((B * MAXC, D), jnp.float32),
    ],
    mesh=_MESH,
    compiler_params=pltpu.CompilerParams(needs_layout_passes=False),
    scratch_types=[
        pltpu.VMEM((16, 16), jnp.int32),
        pltpu.VMEM((OUTSEG,), jnp.int32),
        pltpu.VMEM((NCHUNK, CROWS), jnp.int32),
        pltpu.VMEM((CROWS, D), jnp.float32),
        pltpu.VMEM((CROWS, D), jnp.float32),
        pltpu.VMEM((CROWS, D), jnp.float32),
        pltpu.SemaphoreType.DMA,
        pltpu.SemaphoreType.DMA,
        pltpu.SemaphoreType.DMA,
        pltpu.SemaphoreType.DMA,
        pltpu.SemaphoreType.DMA,
        pltpu.SemaphoreType.DMA,
    ],
)(_gather_body)


@jax.jit
def kernel(hidden_states, boundary_mask):
  hs_flat = hidden_states.reshape(B * L, D)
  mask_i32 = boundary_mask.astype(jnp.int32)
  takef, pm, cnt = _compact_call(mask_i32)
  (chunked_flat,) = _gather_call(hs_flat, takef, cnt)
  chunked = chunked_flat.reshape(B, MAXC, D)
  take_idx = takef.reshape(B, L)[:, :MAXC]
  pad_mask = pm.astype(jnp.bool_)
  return chunked, pad_mask, take_idx
